# baseline (speedup 1.0000x reference)
import functools

import jax
import jax.numpy as jnp
from jax import lax
from jax.experimental import pallas as pl
from jax.experimental.pallas import tpu as pltpu
from jax.experimental.pallas import tpu_sc as plsc

_NC, _NS = 2, 16
_NW = _NC * _NS


def _sc_gather(table, ids, BC, S):
    V, E = table.shape
    spw = BC // _NW
    L = 16
    mesh = plsc.VectorSubcoreMesh(core_axis_name="c", subcore_axis_name="s")

    SH = S // 2
    @functools.partial(
        pl.kernel,
        out_type=jax.ShapeDtypeStruct((BC, S, E // 2), jnp.int32),
        mesh=mesh,
        scratch_types=[
            pltpu.VMEM((spw * S,), jnp.int32),
            pltpu.VMEM((SH, E), jnp.float32),
            pltpu.VMEM((SH, E), jnp.float32),
            pltpu.VMEM((SH, E // 2), jnp.int32),
            pltpu.SemaphoreType.DMA,
            pltpu.SemaphoreType.DMA,
        ],
    )
    def k(table_hbm, idx_hbm, out_hbm, idx_v, rows0, rows1, pack_v,
          sem0, sem1):
        wid = lax.axis_index("s") * _NC + lax.axis_index("c")
        base = wid * spw
        pltpu.sync_copy(idx_hbm.at[pl.ds(base * S, spw * S)], idx_v)

        def start(i, rows, sem):
            pltpu.async_copy(
                table_hbm.at[idx_v.at[pl.ds(i * SH, SH)]], rows, sem)

        def drain(rows, sem):
            pltpu.make_async_copy(
                table_hbm.at[pl.ds(0, SH)], rows, sem).wait()

        def pack_out(i, rows):
            @pl.loop(0, SH)
            def _(r):
                for g in range(E // (2 * L)):
                    a = rows[r, pl.ds(2 * L * g, L)]
                    b = rows[r, pl.ds(2 * L * g + L, L)]
                    ai = lax.bitcast_convert_type(a, jnp.int32)
                    bi = lax.bitcast_convert_type(b, jnp.int32)
                    lo_w = ((ai + 0x8000) >> 16) & 0xFFFF
                    hi_w = (bi + 0x8000) & jnp.int32(-65536)
                    pack_v[r, pl.ds(L * g, L)] = lo_w | hi_w

            pltpu.sync_copy(
                pack_v,
                out_hbm.at[base + i // 2, pl.ds((i % 2) * SH, SH)])

        start(0, rows0, sem0)

        @pl.loop(0, spw)
        def _(p):
            i0 = 2 * p
            start(i0 + 1, rows1, sem1)
            drain(rows0, sem0)
            pack_out(i0, rows0)

            @pl.when(p + 1 < spw)
            def _():
                start(i0 + 2, rows0, sem0)

            drain(rows1, sem1)
            pack_out(i0 + 1, rows1)

    return k(table, ids)


def _tc_body(we_ref, tt_ref, w_ref, csum_ref, dg_ref, out_ref):
    BB, S, E2 = we_ref.shape
    E = 2 * E2
    H = out_ref.shape[2]
    wei = we_ref[...]
    lo = lax.bitcast_convert_type(wei << 16, jnp.float32)
    hi = lax.bitcast_convert_type(wei & jnp.int32(-65536), jnp.float32)
    we2 = jnp.concatenate([lo, hi], axis=-1).reshape(BB * S, E)
    zrow = jnp.zeros((1, E), jnp.float32)
    left = jnp.concatenate([we2[1:], zrow], axis=0)
    right = jnp.concatenate([zrow, we2[:-1]], axis=0)
    r = lax.broadcasted_iota(jnp.int32, (BB * S, 1), 0) % S
    left = jnp.where(r == (S - 1), 0.0, left)
    right = jnp.where(r == 0, 0.0, right)
    x = jnp.dot(we2, w_ref[E:2 * E], preferred_element_type=jnp.float32)
    x = x + jnp.dot(left, w_ref[:E], preferred_element_type=jnp.float32)
    x = x + jnp.dot(right, w_ref[2 * E:], preferred_element_type=jnp.float32)
    x3 = x.reshape(BB, S, H)
    acc = x3 + csum_ref[...][None]
    tt_t = tt_ref[...].astype(jnp.float32).T
    dg = dg_ref[...]
    for bb in range(BB):
        out_ref[bb] = acc[bb] + tt_t[:, bb:bb + 1] * dg


def _tc_chunk_body(buf_ref, we_ref, tt_ref, w_ref, csum_ref, dg_ref, out_ref):
    del buf_ref
    _tc_body(we_ref, tt_ref, w_ref, csum_ref, dg_ref, out_ref)


def _tc_embed_chunk(buf, we3, tt, Wg, csum, dg, off, BB, B):
    BC, S, E2 = we3.shape
    H = csum.shape[1]
    nb = BC // BB
    specs = [
        pl.BlockSpec((BB, S, E2), lambda j: (j, 0, 0)),
        pl.BlockSpec((BB, S), lambda j: (off + j, 0)),
        pl.BlockSpec(Wg.shape, lambda j: (0, 0)),
        pl.BlockSpec((S, H), lambda j: (0, 0)),
        pl.BlockSpec((1, H), lambda j: (0, 0)),
    ]
    if buf is None:
        return pl.pallas_call(
            _tc_body,
            grid=(nb,),
            in_specs=specs,
            out_specs=pl.BlockSpec((BB, S, H), lambda j: (off + j, 0, 0)),
            out_shape=jax.ShapeDtypeStruct((B, S, H), jnp.float32),
        )(we3, tt, Wg, csum, dg)
    return pl.pallas_call(
        _tc_chunk_body,
        grid=(nb,),
        in_specs=[pl.BlockSpec(memory_space=pl.ANY)] + specs,
        out_specs=pl.BlockSpec((BB, S, H), lambda j: (off + j, 0, 0)),
        out_shape=jax.ShapeDtypeStruct((B, S, H), jnp.float32),
        input_output_aliases={0: 0},
    )(buf, we3, tt, Wg, csum, dg)


def kernel(input_ids, token_type_ids, position_ids, word_table, pos_table,
           type_table, W, b, gamma, beta):
    B, S = input_ids.shape
    V, E = word_table.shape
    H = pos_table.shape[1]

    pe = jnp.take(pos_table, position_ids[0], axis=0)
    Wt = W.T * gamma[None, :]
    perm = ([32 * (k // 16) + (k % 16) for k in range(E // 2)]
            + [32 * (k // 16) + 16 + (k % 16) for k in range(E // 2)])
    perm = jnp.array(perm, dtype=jnp.int32)
    Wg = jnp.concatenate(
        [blk[perm] for blk in (Wt[:E], Wt[E:2 * E], Wt[2 * E:])], axis=0)
    csum = (pe + b[None, :] + type_table[0][None, :]) * gamma[None, :] \
        + beta[None, :]
    dg = ((type_table[1] - type_table[0]) * gamma).reshape(1, H)
    tt = token_type_ids.astype(jnp.int32)

    chunks = [32, 64, 64, 96] if B == 256 else [B // 4] * 4
    BB = 8
    ids = input_ids.astype(jnp.int32).reshape(B * S)
    wes = []
    start = 0
    for BC in chunks:
        wes.append(_sc_gather(
            word_table, lax.slice(ids, (start * S,), ((start + BC) * S,)),
            BC, S))
        start += BC
    buf = None
    start = 0
    for BC, we3 in zip(chunks, wes):
        buf = _tc_embed_chunk(buf, we3, tt, Wg, csum, dg, start // BB, BB, B)
        start += BC
    return buf


# speedup vs baseline: 1.0313x; 1.0313x over previous
import functools

import jax
import jax.numpy as jnp
from jax import lax
from jax.experimental import pallas as pl
from jax.experimental.pallas import tpu as pltpu
from jax.experimental.pallas import tpu_sc as plsc

_NC, _NS = 2, 16
_NW = _NC * _NS


def _sc_gather(table, ids, BC, S):
    V, E = table.shape
    spw = BC // _NW
    mesh = plsc.VectorSubcoreMesh(core_axis_name="c", subcore_axis_name="s")

    @functools.partial(
        pl.kernel,
        out_type=jax.ShapeDtypeStruct((BC, S, E), jnp.float32),
        mesh=mesh,
        scratch_types=[
            pltpu.VMEM((spw * S,), jnp.int32),
            pltpu.VMEM((S, E), jnp.float32),
            pltpu.SemaphoreType.DMA,
        ],
    )
    def k(table_hbm, idx_hbm, out_hbm, idx_v, rows_v, sem):
        wid = lax.axis_index("s") * _NC + lax.axis_index("c")
        base = wid * spw
        pltpu.sync_copy(idx_hbm.at[pl.ds(base * S, spw * S)], idx_v)

        @pl.loop(0, spw)
        def _(i):
            pltpu.async_copy(
                table_hbm.at[idx_v.at[pl.ds(i * S, S)]], rows_v, sem
            ).wait()
            pltpu.sync_copy(rows_v, out_hbm.at[base + i])

    return k(table, ids)


def _tc_body(we_ref, tt_ref, w_ref, csum_ref, dg_ref, out_ref):
    BB, S, E = we_ref.shape
    H = out_ref.shape[2]
    we2 = we_ref[...].reshape(BB * S, E)
    zrow = jnp.zeros((1, E), jnp.float32)
    left = jnp.concatenate([we2[1:], zrow], axis=0)
    right = jnp.concatenate([zrow, we2[:-1]], axis=0)
    r = lax.broadcasted_iota(jnp.int32, (BB * S, 1), 0) % S
    left = jnp.where(r == (S - 1), 0.0, left)
    right = jnp.where(r == 0, 0.0, right)
    x = jnp.dot(we2, w_ref[E:2 * E], preferred_element_type=jnp.float32)
    x = x + jnp.dot(left, w_ref[:E], preferred_element_type=jnp.float32)
    x = x + jnp.dot(right, w_ref[2 * E:], preferred_element_type=jnp.float32)
    x3 = x.reshape(BB, S, H)
    acc = x3 + csum_ref[...][None]
    tt_t = tt_ref[...].astype(jnp.float32).T
    dg = dg_ref[...]
    for bb in range(BB):
        out_ref[bb] = acc[bb] + tt_t[:, bb:bb + 1] * dg


def _tc_chunk_body(buf_ref, we_ref, tt_ref, w_ref, csum_ref, dg_ref, out_ref):
    del buf_ref
    _tc_body(we_ref, tt_ref, w_ref, csum_ref, dg_ref, out_ref)


def _tc_embed_chunk(buf, we3, tt, Wg, csum, dg, off, BB, B):
    BC, S, E2 = we3.shape
    H = csum.shape[1]
    nb = BC // BB
    specs = [
        pl.BlockSpec((BB, S, E2), lambda j: (j, 0, 0)),
        pl.BlockSpec((BB, S), lambda j: (off + j, 0)),
        pl.BlockSpec(Wg.shape, lambda j: (0, 0)),
        pl.BlockSpec((S, H), lambda j: (0, 0)),
        pl.BlockSpec((1, H), lambda j: (0, 0)),
    ]
    if buf is None:
        return pl.pallas_call(
            _tc_body,
            grid=(nb,),
            in_specs=specs,
            out_specs=pl.BlockSpec((BB, S, H), lambda j: (off + j, 0, 0)),
            out_shape=jax.ShapeDtypeStruct((B, S, H), jnp.float32),
        )(we3, tt, Wg, csum, dg)
    return pl.pallas_call(
        _tc_chunk_body,
        grid=(nb,),
        in_specs=[pl.BlockSpec(memory_space=pl.ANY)] + specs,
        out_specs=pl.BlockSpec((BB, S, H), lambda j: (off + j, 0, 0)),
        out_shape=jax.ShapeDtypeStruct((B, S, H), jnp.float32),
        input_output_aliases={0: 0},
    )(buf, we3, tt, Wg, csum, dg)


def kernel(input_ids, token_type_ids, position_ids, word_table, pos_table,
           type_table, W, b, gamma, beta):
    B, S = input_ids.shape
    V, E = word_table.shape
    H = pos_table.shape[1]

    pe = jnp.take(pos_table, position_ids[0], axis=0)
    Wg = W.T * gamma[None, :]
    csum = (pe + b[None, :] + type_table[0][None, :]) * gamma[None, :] \
        + beta[None, :]
    dg = ((type_table[1] - type_table[0]) * gamma).reshape(1, H)
    tt = token_type_ids.astype(jnp.int32)

    chunks = [32, 32, 64, 128] if B == 256 else [B // 4] * 4
    BB = 8
    ids = input_ids.astype(jnp.int32).reshape(B * S)
    wes = []
    start = 0
    for BC in chunks:
        wes.append(_sc_gather(
            word_table, lax.slice(ids, (start * S,), ((start + BC) * S,)),
            BC, S))
        start += BC
    buf = None
    start = 0
    for BC, we3 in zip(chunks, wes):
        buf = _tc_embed_chunk(buf, we3, tt, Wg, csum, dg, start // BB, BB, B)
        start += BC
    return buf
